# baseline (speedup 1.0000x reference)
import functools

import jax
import jax.numpy as jnp
from jax import lax
from jax.experimental import pallas as pl
from jax.experimental.pallas import tpu as pltpu
from jax.experimental.pallas import tpu_sc as plsc


@functools.lru_cache(maxsize=None)
def _build(B, S, D, ch_seqs):
    info = plsc.get_sparse_core_info()
    NC, NS, L = info.num_cores, info.num_subcores, info.num_lanes
    NW = NC * NS
    assert B % NW == 0
    seq_per_w = B // NW
    assert seq_per_w % ch_seqs == 0
    nchunk = seq_per_w // ch_seqs
    rows = ch_seqs * S
    nsub, rem = divmod(rows, 128)
    assert D % L == 0

    mesh = plsc.VectorSubcoreMesh(core_axis_name="c", subcore_axis_name="s")

    @functools.partial(
        pl.kernel,
        mesh=mesh,
        compiler_params=pltpu.CompilerParams(use_tc_tiling_on_sc=False),
        out_type=jax.ShapeDtypeStruct((B * S, D), jnp.float32),
        scratch_types=[
            pltpu.VMEM((rows,), jnp.int32),
            pltpu.VMEM((rows, D), jnp.float32),
            pltpu.VMEM((S, D), jnp.float32),
            pltpu.SemaphoreType.DMA,
        ],
    )
    def body(idx_hbm, table_hbm, pe_hbm, out_hbm, idx_v, rows_v, pe_v, sem):
        wid = lax.axis_index("s") * NC + lax.axis_index("c")
        pltpu.sync_copy(pe_hbm, pe_v)
        base_row = wid * (seq_per_w * S)

        def do_chunk(k, carry):
            row0 = pl.multiple_of(base_row + k * rows, 8)
            pltpu.sync_copy(idx_hbm.at[pl.ds(row0, rows)], idx_v)
            cps = []
            for j in range(nsub):
                cps.append(pltpu.async_copy(
                    table_hbm.at[idx_v.at[pl.ds(j * 128, 128)]],
                    rows_v.at[pl.ds(j * 128, 128)], sem))
            if rem:
                cps.append(pltpu.async_copy(
                    table_hbm.at[idx_v.at[pl.ds(nsub * 128, rem)]],
                    rows_v.at[pl.ds(nsub * 128, rem)], sem))
            for cp in cps:
                cp.wait()

            def add_row(r, c2):
                pes = [pe_v[r, pl.ds(q * L, L)] for q in range(D // L)]
                for c in range(ch_seqs):
                    for q in range(D // L):
                        plsc.addupdate(
                            rows_v.at[c * S + r, pl.ds(q * L, L)], pes[q])
                return c2

            lax.fori_loop(0, S, add_row, 0)
            pltpu.sync_copy(rows_v, out_hbm.at[pl.ds(row0, rows)])
            return carry

        lax.fori_loop(0, nchunk, do_chunk, 0)

    return body


def kernel(inputs, table, pos_encoding):
    B, S = inputs.shape
    V, D = table.shape
    idx = inputs.reshape(-1).astype(jnp.int32)
    pe = pos_encoding[:S].astype(jnp.float32)
    out = _build(B, S, D, 4)(idx, table, pe)
    return out.reshape(B, S, D)
